# baseline (speedup 1.0000x reference)
import functools

import jax
import jax.numpy as jnp
from jax import lax
from jax.experimental import pallas as pl
from jax.experimental.pallas import tpu as pltpu
from jax.experimental.pallas import tpu_sc as plsc

PLANES = 16 * 256
T = 4096
OUT = 256 * 31
NW = 32
PPW = PLANES // NW
L = 16
NG = T // L
CPB = 4
NCH = PPW // CPB


def _relayout_plane(in_ref, out_ref, in_off, out_off):
    c31 = lax.iota(jnp.int32, L) * 31

    def base1(g):
        return 496 * (g & 7) + (g >> 3)

    def body(g, lo_ok, hi_ok):
        v = in_ref[pl.ds(in_off + g * L, L)]
        idx1 = (out_off + base1(g)) + c31
        if lo_ok:
            plsc.store_scatter(out_ref, [idx1], v)
        if hi_ok:
            plsc.store_scatter(out_ref, [idx1 + 3967], v)

    lax.fori_loop(0, 8, lambda g, _: (body(g, True, False), 0)[1], 0,
                  unroll=8)
    lax.fori_loop(8, NG - 8, lambda g, _: (body(g, True, True), 0)[1], 0,
                  unroll=8)
    lax.fori_loop(NG - 8, NG, lambda g, _: (body(g, False, True), 0)[1], 0,
                  unroll=8)


def _sc_chunk(x1):
    mesh = plsc.VectorSubcoreMesh(core_axis_name="c", subcore_axis_name="s")

    @functools.partial(
        pl.kernel,
        out_type=jax.ShapeDtypeStruct((PLANES * OUT,), jnp.float32),
        mesh=mesh,
        compiler_params=pltpu.CompilerParams(needs_layout_passes=False),
        scratch_types=[
            pltpu.VMEM((CPB * T,), jnp.float32),
            pltpu.VMEM((CPB * T,), jnp.float32),
            pltpu.VMEM((CPB * OUT,), jnp.float32),
            pltpu.VMEM((CPB * OUT,), jnp.float32),
            pltpu.SemaphoreType.DMA,
            pltpu.SemaphoreType.DMA,
            pltpu.SemaphoreType.DMA,
            pltpu.SemaphoreType.DMA,
        ],
    )
    def k(x_hbm, out_hbm, in0, in1, out0, out1, si0, si1, so0, so1):
        wid = lax.axis_index("s") * 2 + lax.axis_index("c")
        base = wid * PPW
        ins, outs = (in0, in1), (out0, out1)
        sis, sos = (si0, si1), (so0, so1)

        def in_dma(c, b):
            return pltpu.make_async_copy(
                x_hbm.at[pl.ds((base + c * CPB) * T, CPB * T)], ins[b],
                sis[b])

        def out_dma(c, b):
            return pltpu.make_async_copy(
                outs[b], out_hbm.at[pl.ds((base + c * CPB) * OUT, CPB * OUT)],
                sos[b])

        in_dma(0, 0).start()

        def step(c, b):
            @pl.when(c + 1 < NCH)
            def _():
                in_dma(c + 1, 1 - b).start()

            in_dma(c, b).wait()

            @pl.when(c >= 2)
            def _():
                out_dma(c - 2, b).wait()

            for p in range(CPB):
                _relayout_plane(ins[b], outs[b], p * T, p * OUT)
            out_dma(c, b).start()

        def pair(k2, _):
            step(k2 * 2, 0)
            step(k2 * 2 + 1, 1)
            return 0

        lax.fori_loop(0, NCH // 2, pair, 0)
        out_dma(NCH - 2, 0).wait()
        out_dma(NCH - 1, 1).wait()

    return k(x1)


def kernel(x):
    x1 = x.reshape(PLANES * T)
    out1 = _sc_chunk(x1)
    return out1.reshape(16, 256, 256, 31)


# speedup vs baseline: 10.7447x; 10.7447x over previous
import functools

import jax
import jax.numpy as jnp
from jax import lax
from jax.experimental import pallas as pl
from jax.experimental.pallas import tpu as pltpu
from jax.experimental.pallas import tpu_sc as plsc

B = 16
CT = 32
TT = 32
R = 31
TILE = 8 * 128
SLAB = TT * TILE
OSLAB = 2 * TILE
NW = 32
SPW = (B * CT) // NW


def _sc_chunk(x_lin):
    mesh = plsc.VectorSubcoreMesh(core_axis_name="c", subcore_axis_name="s")

    @functools.partial(
        pl.kernel,
        out_type=jax.ShapeDtypeStruct((B * R * CT * OSLAB,), jnp.float32),
        mesh=mesh,
        compiler_params=pltpu.CompilerParams(needs_layout_passes=False),
        scratch_types=[
            pltpu.VMEM((SLAB,), jnp.float32),
            pltpu.VMEM((SLAB,), jnp.float32),
            pltpu.SemaphoreType.DMA,
            pltpu.SemaphoreType.DMA,
            pltpu.SemaphoreType.DMA,
            pltpu.SemaphoreType.DMA,
        ],
    )
    def k(x_hbm, out_hbm, buf0, buf1, si0, si1, so0, so1):
        wid = lax.axis_index("s") * 2 + lax.axis_index("c")
        s0 = wid * SPW
        bufs, sis, sos = (buf0, buf1), (si0, si1), (so0, so1)

        def in_dma(i, p):
            return pltpu.make_async_copy(
                x_hbm.at[pl.ds((s0 + i) * SLAB, SLAB)], bufs[p], sis[p])

        def out_dma(i, r, p):
            s = s0 + i
            b, ct = s >> 5, s & 31
            off = ((b * R + r) * CT + ct) * OSLAB
            return pltpu.make_async_copy(
                bufs[p].at[pl.ds(r * TILE, OSLAB)],
                out_hbm.at[pl.ds(off, OSLAB)], sos[p])

        in_dma(0, 0).start()

        def step(i, p):
            in_dma(i, p).wait()
            for r in range(R):
                out_dma(i, r, p).start()

            @pl.when(i + 1 < SPW)
            def _():
                @pl.when(i >= 1)
                def _():
                    for r in range(R):
                        out_dma(i - 1, r, 1 - p).wait()

                in_dma(i + 1, 1 - p).start()

        def pair(k2, _):
            step(k2 * 2, 0)
            step(k2 * 2 + 1, 1)
            return 0

        lax.fori_loop(0, SPW // 2, pair, 0)
        for r in range(R):
            out_dma(SPW - 2, r, 0).wait()
        for r in range(R):
            out_dma(SPW - 1, r, 1).wait()

    return k(x_lin)


def kernel(x):
    x_lin = x.reshape(B, CT, 8, TT, 128).transpose(0, 1, 3, 2, 4).reshape(-1)
    out_lin = _sc_chunk(x_lin)
    out = (out_lin.reshape(B, R, CT, 2, 8, 128)
           .transpose(0, 2, 4, 3, 5, 1)
           .reshape(16, 256, 256, 31))
    return out
